# baseline (speedup 1.0000x reference)
import functools

import jax
import jax.numpy as jnp
from jax import lax
from jax.experimental import pallas as pl
from jax.experimental.pallas import tpu as pltpu
from jax.experimental.pallas import tpu_sc as plsc

N_NODES = 10000
N_EDGES = 320000
N_GRAPHS = 128
D_FEAT = 128
HIDDEN = 64
NUM_LAYERS = 3
BOTTLENECK = 64
EMB = HIDDEN * NUM_LAYERS

NC = 2
NS = 16
NW = NC * NS
EPW = N_EDGES // NW
EK = 80
NCHUNK = EPW // EK
RPS = 624
RTAIL = N_NODES - NS * RPS

RBLK = 1000
NBLK = N_NODES // RBLK


SK = 128


@functools.lru_cache(maxsize=None)
def _make_edge_agg_ordered(d):
  nsl = d // 16
  nr = N_NODES * nsl
  rps = nr // NS
  nchunk = N_EDGES // SK
  mesh = plsc.VectorSubcoreMesh(core_axis_name="c", subcore_axis_name="s",
                                num_cores=NC, num_subcores=NS)

  @functools.partial(
      pl.kernel,
      out_type=jax.ShapeDtypeStruct((NC, nr, 16), jnp.float32),
      mesh=mesh,
      scratch_types=[
          pltpu.VMEM((SK,), jnp.int32),
          pltpu.VMEM((SK,), jnp.int32),
          pltpu.VMEM((SK,), jnp.int32),
          pltpu.VMEM((SK, 16), jnp.float32),
          pltpu.VMEM_SHARED((nr, 16), jnp.float32),
          pltpu.SemaphoreType.DMA,
      ],
      compiler_params=pltpu.CompilerParams(use_tc_tiling_on_sc=False),
  )
  def edge_agg(h1_hbm, s1_hbm, d1_hbm, h2_hbm, s2_hbm, d2_hbm, zero_hbm,
               out_hbm, si_v, di_v, gi_v, rows_v, agg_sh, sem):
    cid = lax.axis_index("c")
    sid = lax.axis_index("s")
    rbase = sid * rps
    pltpu.sync_copy(zero_hbm.at[pl.ds(rbase, rps)],
                    agg_sh.at[pl.ds(rbase, rps)])
    plsc.subcore_barrier()

    def run(h_hbm, s_hbm, d_hbm):
      def body(c, carry):
        off = c * SK
        pltpu.sync_copy(s_hbm.at[pl.ds(off, SK)], si_v)
        pltpu.sync_copy(d_hbm.at[pl.ds(off, SK)], di_v)
        for q in range(SK // 16):
          sl = pl.ds(q * 16, 16)
          gi_v[sl] = si_v[sl] * nsl + sid
          di_v[sl] = di_v[sl] * nsl + sid
        pltpu.async_copy(h_hbm.at[gi_v], rows_v, sem).wait()
        pltpu.sync_copy(rows_v, agg_sh.at[di_v], add=True)
        return carry

      lax.fori_loop(0, nchunk, body, 0)

    @pl.when(jnp.logical_and(cid == 0, sid < nsl))
    def _():
      run(h1_hbm, s1_hbm, d1_hbm)

    @pl.when(jnp.logical_and(cid == 1, sid < nsl))
    def _():
      run(h2_hbm, s2_hbm, d2_hbm)

    plsc.subcore_barrier()
    pltpu.sync_copy(agg_sh.at[pl.ds(rbase, rps)],
                    out_hbm.at[cid, pl.ds(rbase, rps)])

  return edge_agg


def _mlp_pool_body(hin_ref, agg_ref, batch_ref, w1_ref, b1_ref, w2_ref,
                   b2_ref, h_ref, pooled_ref):
  i = pl.program_id(0)
  zp = hin_ref[...] + agg_ref[...]
  z1 = jnp.maximum(
      jnp.dot(zp, w1_ref[...], preferred_element_type=jnp.float32)
      + b1_ref[...], 0.0)
  h2 = jnp.maximum(
      jnp.dot(z1, w2_ref[...], preferred_element_type=jnp.float32)
      + b2_ref[...], 0.0)
  h_ref[...] = h2
  gids = lax.broadcasted_iota(jnp.int32, (N_GRAPHS, RBLK), 0)
  m = (gids == batch_ref[0]).astype(jnp.float32)
  pooled_blk = jnp.dot(m, h2, preferred_element_type=jnp.float32,
                       precision=lax.Precision.HIGHEST)

  @pl.when(i == 0)
  def _():
    pooled_ref[...] = pooled_blk

  @pl.when(i > 0)
  def _():
    pooled_ref[...] = pooled_ref[...] + pooled_blk


def _mlp_pool(hin, agg, batch3d, w1, b1, w2, b2):
  d = agg.shape[-1]
  return pl.pallas_call(
      _mlp_pool_body,
      grid=(NBLK,),
      in_specs=[
          pl.BlockSpec((RBLK, d), lambda i: (i, 0)),
          pl.BlockSpec((RBLK, d), lambda i: (i, 0)),
          pl.BlockSpec((1, 1, RBLK), lambda i: (i, 0, 0)),
          pl.BlockSpec((d, HIDDEN), lambda i: (0, 0)),
          pl.BlockSpec((1, HIDDEN), lambda i: (0, 0)),
          pl.BlockSpec((HIDDEN, HIDDEN), lambda i: (0, 0)),
          pl.BlockSpec((1, HIDDEN), lambda i: (0, 0)),
      ],
      out_specs=[
          pl.BlockSpec((RBLK, HIDDEN), lambda i: (i, 0)),
          pl.BlockSpec((N_GRAPHS, HIDDEN), lambda i: (0, 0)),
      ],
      out_shape=[
          jax.ShapeDtypeStruct((N_NODES, HIDDEN), jnp.float32),
          jax.ShapeDtypeStruct((N_GRAPHS, HIDDEN), jnp.float32),
      ],
  )(hin, agg, batch3d, w1, b1, w2, b2)


def _heads_body(p0_ref, p1_ref, p2_ref,
                pw1_ref, pb1_ref, pw2_ref, pb2_ref,
                qw1_ref, qb1_ref, gamma_ref, beta_ref, qw2_ref, qb2_ref,
                z_ref, p_ref):
  emb = jnp.concatenate([p0_ref[...], p1_ref[...], p2_ref[...]], axis=1)
  z1 = jnp.maximum(
      jnp.dot(emb, pw1_ref[...], preferred_element_type=jnp.float32)
      + pb1_ref[...], 0.0)
  z = (jnp.dot(z1, pw2_ref[...], preferred_element_type=jnp.float32)
       + pb2_ref[...])
  z_ref[...] = z
  h = (jnp.dot(z, qw1_ref[...], preferred_element_type=jnp.float32)
       + qb1_ref[...])
  mean = jnp.mean(h, axis=0, keepdims=True)
  var = jnp.mean((h - mean) ** 2, axis=0, keepdims=True)
  hn = (h - mean) / jnp.sqrt(var + 1e-5) * gamma_ref[...] + beta_ref[...]
  hr = jnp.maximum(hn, 0.0)
  p_ref[...] = (jnp.dot(hr, qw2_ref[...], preferred_element_type=jnp.float32)
                + qb2_ref[...])


def _heads(p0, p1, p2, params):
  row = lambda a: a.reshape(1, -1)
  return pl.pallas_call(
      _heads_body,
      out_shape=[
          jax.ShapeDtypeStruct((N_GRAPHS, EMB), jnp.float32),
          jax.ShapeDtypeStruct((N_GRAPHS, EMB), jnp.float32),
      ],
  )(p0, p1, p2,
    params['proj_W1'], row(params['proj_b1']),
    params['proj_W2'], row(params['proj_b2']),
    params['pred_W1'], row(params['pred_b1']),
    row(params['bn_gamma']), row(params['bn_beta']),
    params['pred_W2'], row(params['pred_b2']))


def kernel(x1_x, x1_edge_index, x1_batch, x2_x, x2_edge_index, x2_batch,
           params):
  src1 = x1_edge_index[0].astype(jnp.int32)
  dst1 = x1_edge_index[1].astype(jnp.int32)
  src2 = x2_edge_index[0].astype(jnp.int32)
  dst2 = x2_edge_index[1].astype(jnp.int32)
  b3d1 = x1_batch.astype(jnp.int32).reshape(NBLK, 1, RBLK)
  b3d2 = x2_batch.astype(jnp.int32).reshape(NBLK, 1, RBLK)

  h1, h2 = x1_x, x2_x
  pooled1, pooled2 = [], []
  for i in range(NUM_LAYERS):
    d = h1.shape[-1]
    nsl = d // 16
    zeros = jnp.zeros((N_NODES * nsl, 16), jnp.float32)
    aggs = _make_edge_agg_ordered(d)(
        h1.reshape(N_NODES * nsl, 16), src1, dst1,
        h2.reshape(N_NODES * nsl, 16), src2, dst2, zeros)
    agg1 = aggs[0].reshape(N_NODES, d)
    agg2 = aggs[1].reshape(N_NODES, d)
    w1 = params['gin%d_W1' % i]
    b1 = params['gin%d_b1' % i].reshape(1, -1)
    w2 = params['gin%d_W2' % i]
    b2 = params['gin%d_b2' % i].reshape(1, -1)
    h1, p1_i = _mlp_pool(h1, agg1, b3d1, w1, b1, w2, b2)
    h2, p2_i = _mlp_pool(h2, agg2, b3d2, w1, b1, w2, b2)
    pooled1.append(p1_i)
    pooled2.append(p2_i)
  z1, p1 = _heads(pooled1[0], pooled1[1], pooled1[2], params)
  z2, p2 = _heads(pooled2[0], pooled2[1], pooled2[2], params)
  return (p1, z2, p2, z1)


# speedup vs baseline: 2.5895x; 2.5895x over previous
import functools

import jax
import jax.numpy as jnp
from jax import lax
from jax.experimental import pallas as pl
from jax.experimental.pallas import tpu as pltpu
from jax.experimental.pallas import tpu_sc as plsc

N_NODES = 10000
N_EDGES = 320000
N_GRAPHS = 128
D_FEAT = 128
HIDDEN = 64
NUM_LAYERS = 3
BOTTLENECK = 64
EMB = HIDDEN * NUM_LAYERS

NC = 2
NS = 16
NW = NC * NS
EPW = N_EDGES // NW
EK = 80
NCHUNK = EPW // EK
RPS = 624
RTAIL = N_NODES - NS * RPS

RBLK = 1000
NBLK = N_NODES // RBLK


SK = 512


@functools.lru_cache(maxsize=None)
def _make_edge_agg_ordered(d):
  nsl = d // 16
  nr = N_NODES * nsl
  rps = nr // NS
  nchunk = N_EDGES // SK
  mesh = plsc.VectorSubcoreMesh(core_axis_name="c", subcore_axis_name="s",
                                num_cores=NC, num_subcores=NS)

  @functools.partial(
      pl.kernel,
      out_type=jax.ShapeDtypeStruct((NC, nr, 16), jnp.float32),
      mesh=mesh,
      scratch_types=[
          pltpu.VMEM((SK,), jnp.int32),
          pltpu.VMEM((SK,), jnp.int32),
          pltpu.VMEM((SK,), jnp.int32),
          pltpu.VMEM((SK, 16), jnp.float32),
          pltpu.VMEM_SHARED((nr, 16), jnp.float32),
          pltpu.SemaphoreType.DMA,
      ],
      compiler_params=pltpu.CompilerParams(use_tc_tiling_on_sc=False),
  )
  def edge_agg(h1_hbm, s1_hbm, d1_hbm, h2_hbm, s2_hbm, d2_hbm, zero_hbm,
               out_hbm, si_v, di_v, gi_v, rows_v, agg_sh, sem):
    cid = lax.axis_index("c")
    sid = lax.axis_index("s")
    rbase = sid * rps
    pltpu.sync_copy(zero_hbm.at[pl.ds(rbase, rps)],
                    agg_sh.at[pl.ds(rbase, rps)])
    plsc.subcore_barrier()

    def run(h_hbm, s_hbm, d_hbm):
      def body(c, carry):
        off = c * SK
        pltpu.sync_copy(s_hbm.at[pl.ds(off, SK)], si_v)
        pltpu.sync_copy(d_hbm.at[pl.ds(off, SK)], di_v)
        for q in range(SK // 16):
          sl = pl.ds(q * 16, 16)
          gi_v[sl] = si_v[sl] * nsl + sid
          di_v[sl] = di_v[sl] * nsl + sid
        pltpu.async_copy(h_hbm.at[gi_v], rows_v, sem).wait()
        pltpu.sync_copy(rows_v, agg_sh.at[di_v], add=True)
        return carry

      lax.fori_loop(0, nchunk, body, 0)

    @pl.when(jnp.logical_and(cid == 0, sid < nsl))
    def _():
      run(h1_hbm, s1_hbm, d1_hbm)

    @pl.when(jnp.logical_and(cid == 1, sid < nsl))
    def _():
      run(h2_hbm, s2_hbm, d2_hbm)

    plsc.subcore_barrier()
    pltpu.sync_copy(agg_sh.at[pl.ds(rbase, rps)],
                    out_hbm.at[cid, pl.ds(rbase, rps)])

  return edge_agg


def _mlp_pool_body(hin_ref, agg_ref, batch_ref, w1_ref, b1_ref, w2_ref,
                   b2_ref, h_ref, pooled_ref):
  i = pl.program_id(0)
  zp = hin_ref[...] + agg_ref[...]
  z1 = jnp.maximum(
      jnp.dot(zp, w1_ref[...], preferred_element_type=jnp.float32)
      + b1_ref[...], 0.0)
  h2 = jnp.maximum(
      jnp.dot(z1, w2_ref[...], preferred_element_type=jnp.float32)
      + b2_ref[...], 0.0)
  h_ref[...] = h2
  gids = lax.broadcasted_iota(jnp.int32, (N_GRAPHS, RBLK), 0)
  m = (gids == batch_ref[0]).astype(jnp.float32)
  pooled_blk = jnp.dot(m, h2, preferred_element_type=jnp.float32,
                       precision=lax.Precision.HIGHEST)

  @pl.when(i == 0)
  def _():
    pooled_ref[...] = pooled_blk

  @pl.when(i > 0)
  def _():
    pooled_ref[...] = pooled_ref[...] + pooled_blk


def _mlp_pool(hin, agg, batch3d, w1, b1, w2, b2):
  d = agg.shape[-1]
  return pl.pallas_call(
      _mlp_pool_body,
      grid=(NBLK,),
      in_specs=[
          pl.BlockSpec((RBLK, d), lambda i: (i, 0)),
          pl.BlockSpec((RBLK, d), lambda i: (i, 0)),
          pl.BlockSpec((1, 1, RBLK), lambda i: (i, 0, 0)),
          pl.BlockSpec((d, HIDDEN), lambda i: (0, 0)),
          pl.BlockSpec((1, HIDDEN), lambda i: (0, 0)),
          pl.BlockSpec((HIDDEN, HIDDEN), lambda i: (0, 0)),
          pl.BlockSpec((1, HIDDEN), lambda i: (0, 0)),
      ],
      out_specs=[
          pl.BlockSpec((RBLK, HIDDEN), lambda i: (i, 0)),
          pl.BlockSpec((N_GRAPHS, HIDDEN), lambda i: (0, 0)),
      ],
      out_shape=[
          jax.ShapeDtypeStruct((N_NODES, HIDDEN), jnp.float32),
          jax.ShapeDtypeStruct((N_GRAPHS, HIDDEN), jnp.float32),
      ],
  )(hin, agg, batch3d, w1, b1, w2, b2)


def _heads_body(p0_ref, p1_ref, p2_ref,
                pw1_ref, pb1_ref, pw2_ref, pb2_ref,
                qw1_ref, qb1_ref, gamma_ref, beta_ref, qw2_ref, qb2_ref,
                z_ref, p_ref):
  emb = jnp.concatenate([p0_ref[...], p1_ref[...], p2_ref[...]], axis=1)
  z1 = jnp.maximum(
      jnp.dot(emb, pw1_ref[...], preferred_element_type=jnp.float32)
      + pb1_ref[...], 0.0)
  z = (jnp.dot(z1, pw2_ref[...], preferred_element_type=jnp.float32)
       + pb2_ref[...])
  z_ref[...] = z
  h = (jnp.dot(z, qw1_ref[...], preferred_element_type=jnp.float32)
       + qb1_ref[...])
  mean = jnp.mean(h, axis=0, keepdims=True)
  var = jnp.mean((h - mean) ** 2, axis=0, keepdims=True)
  hn = (h - mean) / jnp.sqrt(var + 1e-5) * gamma_ref[...] + beta_ref[...]
  hr = jnp.maximum(hn, 0.0)
  p_ref[...] = (jnp.dot(hr, qw2_ref[...], preferred_element_type=jnp.float32)
                + qb2_ref[...])


def _heads(p0, p1, p2, params):
  row = lambda a: a.reshape(1, -1)
  return pl.pallas_call(
      _heads_body,
      out_shape=[
          jax.ShapeDtypeStruct((N_GRAPHS, EMB), jnp.float32),
          jax.ShapeDtypeStruct((N_GRAPHS, EMB), jnp.float32),
      ],
  )(p0, p1, p2,
    params['proj_W1'], row(params['proj_b1']),
    params['proj_W2'], row(params['proj_b2']),
    params['pred_W1'], row(params['pred_b1']),
    row(params['bn_gamma']), row(params['bn_beta']),
    params['pred_W2'], row(params['pred_b2']))


def kernel(x1_x, x1_edge_index, x1_batch, x2_x, x2_edge_index, x2_batch,
           params):
  src1 = x1_edge_index[0].astype(jnp.int32)
  dst1 = x1_edge_index[1].astype(jnp.int32)
  src2 = x2_edge_index[0].astype(jnp.int32)
  dst2 = x2_edge_index[1].astype(jnp.int32)
  b3d1 = x1_batch.astype(jnp.int32).reshape(NBLK, 1, RBLK)
  b3d2 = x2_batch.astype(jnp.int32).reshape(NBLK, 1, RBLK)

  h1, h2 = x1_x, x2_x
  pooled1, pooled2 = [], []
  for i in range(NUM_LAYERS):
    d = h1.shape[-1]
    nsl = d // 16
    zeros = jnp.zeros((N_NODES * nsl, 16), jnp.float32)
    aggs = _make_edge_agg_ordered(d)(
        h1.reshape(N_NODES * nsl, 16), src1, dst1,
        h2.reshape(N_NODES * nsl, 16), src2, dst2, zeros)
    agg1 = aggs[0].reshape(N_NODES, d)
    agg2 = aggs[1].reshape(N_NODES, d)
    w1 = params['gin%d_W1' % i]
    b1 = params['gin%d_b1' % i].reshape(1, -1)
    w2 = params['gin%d_W2' % i]
    b2 = params['gin%d_b2' % i].reshape(1, -1)
    h1, p1_i = _mlp_pool(h1, agg1, b3d1, w1, b1, w2, b2)
    h2, p2_i = _mlp_pool(h2, agg2, b3d2, w1, b1, w2, b2)
    pooled1.append(p1_i)
    pooled2.append(p2_i)
  z1, p1 = _heads(pooled1[0], pooled1[1], pooled1[2], params)
  z2, p2 = _heads(pooled2[0], pooled2[1], pooled2[2], params)
  return (p1, z2, p2, z1)


# speedup vs baseline: 4.4789x; 1.7297x over previous
import functools

import jax
import jax.numpy as jnp
from jax import lax
from jax.experimental import pallas as pl
from jax.experimental.pallas import tpu as pltpu
from jax.experimental.pallas import tpu_sc as plsc

N_NODES = 10000
N_EDGES = 320000
N_GRAPHS = 128
D_FEAT = 128
HIDDEN = 64
NUM_LAYERS = 3
BOTTLENECK = 64
EMB = HIDDEN * NUM_LAYERS

NC = 2
NS = 16
NW = NC * NS
EPW = N_EDGES // NW
EK = 80
NCHUNK = EPW // EK
RPS = 624
RTAIL = N_NODES - NS * RPS

RBLK = 1000
NBLK = N_NODES // RBLK


SK = 2000


@functools.lru_cache(maxsize=None)
def _make_edge_agg_ordered(d):
  nsl = d // 16
  nr = N_NODES * nsl
  rps = nr // NS
  nchunk = N_EDGES // SK
  mesh = plsc.VectorSubcoreMesh(core_axis_name="c", subcore_axis_name="s",
                                num_cores=NC, num_subcores=NS)

  @functools.partial(
      pl.kernel,
      out_type=jax.ShapeDtypeStruct((NC, nr, 16), jnp.float32),
      mesh=mesh,
      scratch_types=[
          pltpu.VMEM((SK,), jnp.int32),
          pltpu.VMEM((SK,), jnp.int32),
          pltpu.VMEM((SK,), jnp.int32),
          pltpu.VMEM((SK, 16), jnp.float32),
          pltpu.VMEM_SHARED((nr, 16), jnp.float32),
          pltpu.SemaphoreType.DMA,
      ],
      compiler_params=pltpu.CompilerParams(use_tc_tiling_on_sc=False),
  )
  def edge_agg(h1_hbm, s1_hbm, d1_hbm, h2_hbm, s2_hbm, d2_hbm, zero_hbm,
               out_hbm, si_v, di_v, gi_v, rows_v, agg_sh, sem):
    cid = lax.axis_index("c")
    sid = lax.axis_index("s")
    rbase = sid * rps
    pltpu.sync_copy(zero_hbm.at[pl.ds(rbase, rps)],
                    agg_sh.at[pl.ds(rbase, rps)])
    plsc.subcore_barrier()

    def run(h_hbm, s_hbm, d_hbm):
      def body(c, carry):
        off = c * SK
        pltpu.sync_copy(s_hbm.at[pl.ds(off, SK)], si_v)
        pltpu.sync_copy(d_hbm.at[pl.ds(off, SK)], di_v)
        for q in range(SK // 16):
          sl = pl.ds(q * 16, 16)
          gi_v[sl] = si_v[sl] * nsl + sid
          di_v[sl] = di_v[sl] * nsl + sid
        pltpu.async_copy(h_hbm.at[gi_v], rows_v, sem).wait()
        pltpu.sync_copy(rows_v, agg_sh.at[di_v], add=True)
        return carry

      lax.fori_loop(0, nchunk, body, 0)

    @pl.when(jnp.logical_and(cid == 0, sid < nsl))
    def _():
      run(h1_hbm, s1_hbm, d1_hbm)

    @pl.when(jnp.logical_and(cid == 1, sid < nsl))
    def _():
      run(h2_hbm, s2_hbm, d2_hbm)

    plsc.subcore_barrier()
    pltpu.sync_copy(agg_sh.at[pl.ds(rbase, rps)],
                    out_hbm.at[cid, pl.ds(rbase, rps)])

  return edge_agg


def _mlp_pool_body(hin_ref, agg_ref, batch_ref, w1_ref, b1_ref, w2_ref,
                   b2_ref, h_ref, pooled_ref):
  i = pl.program_id(0)
  zp = hin_ref[...] + agg_ref[...]
  z1 = jnp.maximum(
      jnp.dot(zp, w1_ref[...], preferred_element_type=jnp.float32)
      + b1_ref[...], 0.0)
  h2 = jnp.maximum(
      jnp.dot(z1, w2_ref[...], preferred_element_type=jnp.float32)
      + b2_ref[...], 0.0)
  h_ref[...] = h2
  gids = lax.broadcasted_iota(jnp.int32, (N_GRAPHS, RBLK), 0)
  m = (gids == batch_ref[0]).astype(jnp.float32)
  pooled_blk = jnp.dot(m, h2, preferred_element_type=jnp.float32,
                       precision=lax.Precision.HIGHEST)

  @pl.when(i == 0)
  def _():
    pooled_ref[...] = pooled_blk

  @pl.when(i > 0)
  def _():
    pooled_ref[...] = pooled_ref[...] + pooled_blk


def _mlp_pool(hin, agg, batch3d, w1, b1, w2, b2):
  d = agg.shape[-1]
  return pl.pallas_call(
      _mlp_pool_body,
      grid=(NBLK,),
      in_specs=[
          pl.BlockSpec((RBLK, d), lambda i: (i, 0)),
          pl.BlockSpec((RBLK, d), lambda i: (i, 0)),
          pl.BlockSpec((1, 1, RBLK), lambda i: (i, 0, 0)),
          pl.BlockSpec((d, HIDDEN), lambda i: (0, 0)),
          pl.BlockSpec((1, HIDDEN), lambda i: (0, 0)),
          pl.BlockSpec((HIDDEN, HIDDEN), lambda i: (0, 0)),
          pl.BlockSpec((1, HIDDEN), lambda i: (0, 0)),
      ],
      out_specs=[
          pl.BlockSpec((RBLK, HIDDEN), lambda i: (i, 0)),
          pl.BlockSpec((N_GRAPHS, HIDDEN), lambda i: (0, 0)),
      ],
      out_shape=[
          jax.ShapeDtypeStruct((N_NODES, HIDDEN), jnp.float32),
          jax.ShapeDtypeStruct((N_GRAPHS, HIDDEN), jnp.float32),
      ],
  )(hin, agg, batch3d, w1, b1, w2, b2)


def _heads_body(p0_ref, p1_ref, p2_ref,
                pw1_ref, pb1_ref, pw2_ref, pb2_ref,
                qw1_ref, qb1_ref, gamma_ref, beta_ref, qw2_ref, qb2_ref,
                z_ref, p_ref):
  emb = jnp.concatenate([p0_ref[...], p1_ref[...], p2_ref[...]], axis=1)
  z1 = jnp.maximum(
      jnp.dot(emb, pw1_ref[...], preferred_element_type=jnp.float32)
      + pb1_ref[...], 0.0)
  z = (jnp.dot(z1, pw2_ref[...], preferred_element_type=jnp.float32)
       + pb2_ref[...])
  z_ref[...] = z
  h = (jnp.dot(z, qw1_ref[...], preferred_element_type=jnp.float32)
       + qb1_ref[...])
  mean = jnp.mean(h, axis=0, keepdims=True)
  var = jnp.mean((h - mean) ** 2, axis=0, keepdims=True)
  hn = (h - mean) / jnp.sqrt(var + 1e-5) * gamma_ref[...] + beta_ref[...]
  hr = jnp.maximum(hn, 0.0)
  p_ref[...] = (jnp.dot(hr, qw2_ref[...], preferred_element_type=jnp.float32)
                + qb2_ref[...])


def _heads(p0, p1, p2, params):
  row = lambda a: a.reshape(1, -1)
  return pl.pallas_call(
      _heads_body,
      out_shape=[
          jax.ShapeDtypeStruct((N_GRAPHS, EMB), jnp.float32),
          jax.ShapeDtypeStruct((N_GRAPHS, EMB), jnp.float32),
      ],
  )(p0, p1, p2,
    params['proj_W1'], row(params['proj_b1']),
    params['proj_W2'], row(params['proj_b2']),
    params['pred_W1'], row(params['pred_b1']),
    row(params['bn_gamma']), row(params['bn_beta']),
    params['pred_W2'], row(params['pred_b2']))


def kernel(x1_x, x1_edge_index, x1_batch, x2_x, x2_edge_index, x2_batch,
           params):
  src1 = x1_edge_index[0].astype(jnp.int32)
  dst1 = x1_edge_index[1].astype(jnp.int32)
  src2 = x2_edge_index[0].astype(jnp.int32)
  dst2 = x2_edge_index[1].astype(jnp.int32)
  b3d1 = x1_batch.astype(jnp.int32).reshape(NBLK, 1, RBLK)
  b3d2 = x2_batch.astype(jnp.int32).reshape(NBLK, 1, RBLK)

  h1, h2 = x1_x, x2_x
  pooled1, pooled2 = [], []
  for i in range(NUM_LAYERS):
    d = h1.shape[-1]
    nsl = d // 16
    zeros = jnp.zeros((N_NODES * nsl, 16), jnp.float32)
    aggs = _make_edge_agg_ordered(d)(
        h1.reshape(N_NODES * nsl, 16), src1, dst1,
        h2.reshape(N_NODES * nsl, 16), src2, dst2, zeros)
    agg1 = aggs[0].reshape(N_NODES, d)
    agg2 = aggs[1].reshape(N_NODES, d)
    w1 = params['gin%d_W1' % i]
    b1 = params['gin%d_b1' % i].reshape(1, -1)
    w2 = params['gin%d_W2' % i]
    b2 = params['gin%d_b2' % i].reshape(1, -1)
    h1, p1_i = _mlp_pool(h1, agg1, b3d1, w1, b1, w2, b2)
    h2, p2_i = _mlp_pool(h2, agg2, b3d2, w1, b1, w2, b2)
    pooled1.append(p1_i)
    pooled2.append(p2_i)
  z1, p1 = _heads(pooled1[0], pooled1[1], pooled1[2], params)
  z2, p2 = _heads(pooled2[0], pooled2[1], pooled2[2], params)
  return (p1, z2, p2, z1)
